# baseline (speedup 1.0000x reference)
import functools

import jax
import jax.numpy as jnp
from jax.experimental import pallas as pl
from jax.experimental.pallas import tpu as pltpu

_F32 = jnp.float32
_PT = 8


def _zero_border(o_ref, C, H, Wl, d):
    o_ref[:, 0:_PT, :] = jnp.zeros((C, _PT, Wl + 2 * d), _F32)
    o_ref[:, _PT + H:_PT + H + _PT, :] = jnp.zeros((C, _PT, Wl + 2 * d), _F32)
    o_ref[:, :, 0:d] = jnp.zeros((C, 2 * _PT + H, d), _F32)
    o_ref[:, :, Wl + d:Wl + 2 * d] = jnp.zeros((C, 2 * _PT + H, d), _F32)


def _shl(v, d):
    return jnp.concatenate([v[..., d:], v[..., :d]], axis=-1)


def _shr(v, d):
    return jnp.concatenate([v[..., -d:], v[..., :-d]], axis=-1)


def _conv3x3_body(Cin, Cout, H, Wl, TH, din, pad_out, mask_inf,
                  x_ref, w_ref, b_ref, o_ref, a_ref):
    oy, ox = (_PT, din) if pad_out else (0, 0)
    if pad_out:
        _zero_border(o_ref, Cout, H, Wl, din)

    def strip(s, _):
        r0 = s * TH
        a_ref[...] = jnp.broadcast_to(b_ref[...][:, None, :], (Cout, TH, Wl))

        def per_in(i, _):
            slab = x_ref[i, pl.ds(r0, TH + 9), :]
            acc = jnp.zeros((Cout, TH, Wl), _F32)
            for dy in range(3):
                for dx in range(3):
                    wrow = w_ref[i * 9 + dy * 3 + dx]
                    xs = slab[7 + dy:7 + dy + TH, dx * din:dx * din + Wl]
                    acc = acc + wrow[:, None, :] * xs[None, :, :]
            a_ref[...] = a_ref[...] + acc
            return 0

        jax.lax.fori_loop(0, Cin, per_in, 0)
        v = a_ref[...]
        if mask_inf:
            v = jnp.where(jnp.isinf(v), 0.0, v)
        o_ref[:, pl.ds(oy + r0, TH), pl.ds(ox, Wl)] = v
        return 0

    jax.lax.fori_loop(0, H // TH, strip, 0)


def _conv3x3(xp, w, b, TH, din=1, pad_out=False, mask_inf=False):
    Cout, Cin = w.shape[0], w.shape[1]
    H, Wl = xp.shape[1] - 2 * _PT, xp.shape[2] - 2 * din
    w3 = jnp.broadcast_to(
        w.transpose(1, 2, 3, 0).reshape(Cin * 9, Cout)[:, :, None],
        (Cin * 9, Cout, Wl))
    b2 = jnp.broadcast_to(b[:, None], (Cout, Wl))
    oshape = ((Cout, 2 * _PT + H, Wl + 2 * din) if pad_out
              else (Cout, H, Wl))
    fn = functools.partial(_conv3x3_body, Cin, Cout, H, Wl, TH, din, pad_out,
                           mask_inf)
    return pl.pallas_call(
        fn,
        out_shape=jax.ShapeDtypeStruct(oshape, _F32),
        scratch_shapes=[pltpu.VMEM((Cout, TH, Wl), _F32)],
    )(xp, w3, b2)


def _pmerge_body(C, D, H, Wl, TS, din, x_ref, g_ref, bt_ref, l_ref,
                 o_ref, n_ref):
    inv = 1.0 / (4 * C)

    def strip(s, _):
        xr = x_ref[:, pl.ds(2 * s * TS, 2 * TS), :].reshape(C, TS, 2, Wl)
        A = jnp.concatenate([xr[:, :, 0, :], xr[:, :, 1, :]], axis=0)
        lane = jax.lax.broadcasted_iota(jnp.int32, (TS, Wl), 1)
        m = ((lane // din) % 2) == 0

        def pair(v):
            return v + jnp.where(m, _shl(v, din), _shr(v, din))

        s1 = jnp.sum(A, axis=0)
        q1 = jnp.sum(A * A, axis=0)
        mu = pair(s1) * inv
        var = pair(q1) * inv - mu * mu
        rstd = jax.lax.rsqrt(var + 1e-5)
        n_ref[...] = ((A - mu[None]) * rstd[None] * g_ref[...][:, None, :]
                      + bt_ref[...][:, None, :])
        rows = pl.ds(s * TS, TS)

        dB = 6
        for d0 in range(0, D, dB):
            def per_in(g, acc):
                return acc + (l_ref[g, d0:d0 + dB][:, None, :]
                              * n_ref[g][None])

            t = jax.lax.fori_loop(0, 2 * C, per_in,
                                  jnp.zeros((dB, TS, Wl), _F32))
            o_ref[d0:d0 + dB, rows, :] = (
                t + jnp.where(m[None], _shl(t, din), _shr(t, din)))
        return 0

    jax.lax.fori_loop(0, (H // 2) // TS, strip, 0)


def _pmerge(x, gamma, beta, lin, TS, din):
    C, H, Wl = x.shape
    D = lin.shape[1]
    pat = ((jnp.arange(Wl) // din) % 2) == 0
    g4 = gamma.reshape(2, 2, C)
    gfull = jnp.where(pat[None, :], g4[0].reshape(2 * C)[:, None],
                      g4[1].reshape(2 * C)[:, None])
    b4 = beta.reshape(2, 2, C)
    btfull = jnp.where(pat[None, :], b4[0].reshape(2 * C)[:, None],
                       b4[1].reshape(2 * C)[:, None])
    l4 = lin.reshape(2, 2, C, D)
    lfull = jnp.where(pat[None, None, :],
                      l4[0].reshape(2 * C, D)[:, :, None],
                      l4[1].reshape(2 * C, D)[:, :, None])
    fn = functools.partial(_pmerge_body, C, D, H, Wl, TS, din)
    return pl.pallas_call(
        fn,
        out_shape=jax.ShapeDtypeStruct((D, H // 2, Wl), _F32),
        scratch_shapes=[pltpu.VMEM((2 * C, TS, Wl), _F32)],
    )(x, gfull, btfull, lfull)


def _upsample_body(Cin, Cmid, H, Wl, TS, din, pad_out,
                   x_ref, w_ref, b_ref, o_ref):
    Co = Cmid // 4
    dout = din // 2
    oy, ox = (_PT, dout) if pad_out else (0, 0)
    if pad_out:
        _zero_border(o_ref, Co, 2 * H, Wl, dout)

    def strip(s, _):
        rows = pl.ds(s * TS, TS)
        lane = jax.lax.broadcasted_iota(jnp.int32, (TS, Wl), 1)
        half = ((lane // dout) % 2) == 0

        for co in range(Co):
            def per_in(i, acc):
                return acc + (w_ref[i, 4 * co:4 * co + 4][:, None, :]
                              * x_ref[i, rows, :][None])

            q = jax.lax.fori_loop(
                0, Cin, per_in,
                jnp.broadcast_to(b_ref[4 * co:4 * co + 4][:, None, :],
                                 (4, TS, Wl)))
            band0 = jnp.where(half, q[0], q[1])
            band1 = jnp.where(half, q[2], q[3])
            full = jnp.stack([band0, band1], axis=1).reshape(2 * TS, Wl)
            o_ref[co, pl.ds(oy + 2 * s * TS, 2 * TS), pl.ds(ox, Wl)] = full
        return 0

    jax.lax.fori_loop(0, H // TS, strip, 0)


def _upsample(x, w, b, TS, din, pad_out=False):
    Cmid, Cin = w.shape[0], w.shape[1]
    C, H, Wl = x.shape
    Co = Cmid // 4
    dout = din // 2
    w3 = jnp.broadcast_to(
        w.reshape(Cmid, Cin).transpose(1, 0)[:, :, None], (Cin, Cmid, Wl))
    b2 = jnp.broadcast_to(b[:, None], (Cmid, Wl))
    oshape = ((Co, 2 * _PT + 2 * H, Wl + 2 * dout) if pad_out
              else (Co, 2 * H, Wl))
    fn = functools.partial(_upsample_body, Cin, Cmid, H, Wl, TS, din, pad_out)
    return pl.pallas_call(
        fn,
        out_shape=jax.ShapeDtypeStruct(oshape, _F32),
    )(x, w3, b2)


def kernel(x, con11_w, con11_b, con1_w, con1_b, con3_w, con3_b, con5_w,
           con5_b, dm1_gamma, dm1_beta, dm1_lin, dm2_gamma, dm2_beta,
           dm2_lin, up2_w, up2_b, up1_w, up1_b):
    xp = jnp.pad(x[0], ((0, 0), (_PT, _PT), (1, 1)))

    y1 = _conv3x3(xp, con11_w, con11_b, TH=48, mask_inf=True)
    d1 = _pmerge(y1, dm1_gamma, dm1_beta, dm1_lin, TS=24, din=1)
    d2 = _pmerge(d1, dm2_gamma, dm2_beta, dm2_lin, TS=16, din=2)
    u2 = _upsample(d2, up2_w, up2_b, TS=16, din=4, pad_out=True)
    c1 = _conv3x3(u2, con1_w, con1_b, TH=24, din=2)
    u1 = _upsample(c1, up1_w, up1_b, TS=24, din=2, pad_out=True)
    c3 = _conv3x3(u1, con3_w, con3_b, TH=48, pad_out=True)
    c4 = _conv3x3(c3, con3_w, con3_b, TH=48, pad_out=True)
    c5 = _conv3x3(c4, con5_w, con5_b, TH=48)
    return c5.reshape(1, 1, 384, 384)


# speedup vs baseline: 1.1263x; 1.1263x over previous
import functools

import jax
import jax.numpy as jnp
from jax.experimental import pallas as pl
from jax.experimental.pallas import tpu as pltpu

_F32 = jnp.float32
_PT = 8


def _zero_border(o_ref, C, H, Wl, d):
    o_ref[:, 0:_PT, :] = jnp.zeros((C, _PT, Wl + 2 * d), _F32)
    o_ref[:, _PT + H:_PT + H + _PT, :] = jnp.zeros((C, _PT, Wl + 2 * d), _F32)
    o_ref[:, :, 0:d] = jnp.zeros((C, 2 * _PT + H, d), _F32)
    o_ref[:, :, Wl + d:Wl + 2 * d] = jnp.zeros((C, 2 * _PT + H, d), _F32)


def _shl(v, d):
    return jnp.concatenate([v[..., d:], v[..., :d]], axis=-1)


def _shr(v, d):
    return jnp.concatenate([v[..., -d:], v[..., :-d]], axis=-1)


def _conv3x3_body(Cin, Cout, H, Wl, TH, din, pad_out, mask_inf,
                  x_ref, w_ref, b_ref, o_ref, a_ref):
    oy, ox = (_PT, din) if pad_out else (0, 0)
    if pad_out:
        _zero_border(o_ref, Cout, H, Wl, din)

    def strip(s, _):
        r0 = s * TH
        a_ref[...] = jnp.broadcast_to(b_ref[...][:, None, :], (Cout, TH, Wl))

        def per_in(i, _):
            slab = x_ref[i, pl.ds(r0, TH + 9), :]
            acc = jnp.zeros((Cout, TH, Wl), _F32)
            for dy in range(3):
                for dx in range(3):
                    wrow = w_ref[i * 9 + dy * 3 + dx]
                    xs = slab[7 + dy:7 + dy + TH, dx * din:dx * din + Wl]
                    acc = acc + wrow[:, None, :] * xs[None, :, :]
            a_ref[...] = a_ref[...] + acc
            return 0

        jax.lax.fori_loop(0, Cin, per_in, 0)
        v = a_ref[...]
        if mask_inf:
            v = jnp.where(jnp.isinf(v), 0.0, v)
        o_ref[:, pl.ds(oy + r0, TH), pl.ds(ox, Wl)] = v
        return 0

    jax.lax.fori_loop(0, H // TH, strip, 0)


def _conv3x3(xp, w, b, TH, din=1, pad_out=False, mask_inf=False):
    Cout, Cin = w.shape[0], w.shape[1]
    H, Wl = xp.shape[1] - 2 * _PT, xp.shape[2] - 2 * din
    w3 = jnp.broadcast_to(
        w.transpose(1, 2, 3, 0).reshape(Cin * 9, Cout)[:, :, None],
        (Cin * 9, Cout, Wl))
    b2 = jnp.broadcast_to(b[:, None], (Cout, Wl))
    oshape = ((Cout, 2 * _PT + H, Wl + 2 * din) if pad_out
              else (Cout, H, Wl))
    fn = functools.partial(_conv3x3_body, Cin, Cout, H, Wl, TH, din, pad_out,
                           mask_inf)
    return pl.pallas_call(
        fn,
        out_shape=jax.ShapeDtypeStruct(oshape, _F32),
        scratch_shapes=[pltpu.VMEM((Cout, TH, Wl), _F32)],
    )(xp, w3, b2)


def _pmerge_body(C, D, H, Wl, TS, din, x_ref, g_ref, bt_ref, l_ref,
                 o_ref, n_ref):
    inv = 1.0 / (4 * C)

    def strip(s, _):
        xr = x_ref[:, pl.ds(2 * s * TS, 2 * TS), :].reshape(C, TS, 2, Wl)
        A = jnp.concatenate([xr[:, :, 0, :], xr[:, :, 1, :]], axis=0)
        lane = jax.lax.broadcasted_iota(jnp.int32, (TS, Wl), 1)
        m = ((lane // din) % 2) == 0

        def pair(v):
            return v + jnp.where(m, _shl(v, din), _shr(v, din))

        s1 = jnp.sum(A, axis=0)
        q1 = jnp.sum(A * A, axis=0)
        mu = pair(s1) * inv
        var = pair(q1) * inv - mu * mu
        rstd = jax.lax.rsqrt(var + 1e-5)
        n_ref[...] = ((A - mu[None]) * rstd[None] * g_ref[...][:, None, :]
                      + bt_ref[...][:, None, :])
        rows = pl.ds(s * TS, TS)
        o_ref[:, rows, :] = jnp.zeros((D, TS, Wl), _F32)

        def per_in(gb, _):
            g0 = gb * 4
            upd = (l_ref[g0][:, None, :] * n_ref[g0][None]
                   + l_ref[g0 + 1][:, None, :] * n_ref[g0 + 1][None]
                   + l_ref[g0 + 2][:, None, :] * n_ref[g0 + 2][None]
                   + l_ref[g0 + 3][:, None, :] * n_ref[g0 + 3][None])
            o_ref[:, rows, :] = o_ref[:, rows, :] + upd
            return 0

        jax.lax.fori_loop(0, (2 * C) // 4, per_in, 0)
        t = o_ref[:, rows, :]
        o_ref[:, rows, :] = t + jnp.where(m[None], _shl(t, din), _shr(t, din))
        return 0

    jax.lax.fori_loop(0, (H // 2) // TS, strip, 0)


def _pmerge(x, gamma, beta, lin, TS, din):
    C, H, Wl = x.shape
    D = lin.shape[1]
    pat = ((jnp.arange(Wl) // din) % 2) == 0
    g4 = gamma.reshape(2, 2, C)
    gfull = jnp.where(pat[None, :], g4[0].reshape(2 * C)[:, None],
                      g4[1].reshape(2 * C)[:, None])
    b4 = beta.reshape(2, 2, C)
    btfull = jnp.where(pat[None, :], b4[0].reshape(2 * C)[:, None],
                       b4[1].reshape(2 * C)[:, None])
    l4 = lin.reshape(2, 2, C, D)
    lfull = jnp.where(pat[None, None, :],
                      l4[0].reshape(2 * C, D)[:, :, None],
                      l4[1].reshape(2 * C, D)[:, :, None])
    fn = functools.partial(_pmerge_body, C, D, H, Wl, TS, din)
    return pl.pallas_call(
        fn,
        out_shape=jax.ShapeDtypeStruct((D, H // 2, Wl), _F32),
        scratch_shapes=[pltpu.VMEM((2 * C, TS, Wl), _F32)],
    )(x, gfull, btfull, lfull)


def _upsample_body(Cin, Cmid, H, Wl, TS, din, pad_out,
                   x_ref, w_ref, b_ref, o_ref, m_ref):
    Co = Cmid // 4
    dout = din // 2
    oy, ox = (_PT, dout) if pad_out else (0, 0)
    if pad_out:
        _zero_border(o_ref, Co, 2 * H, Wl, dout)

    def strip(s, _):
        rows = pl.ds(s * TS, TS)
        m_ref[...] = jnp.broadcast_to(b_ref[...][:, None, :], (Cmid, TS, Wl))

        def per_in(ib, _):
            i0 = ib * 4
            upd = (w_ref[i0][:, None, :] * x_ref[i0, rows, :][None]
                   + w_ref[i0 + 1][:, None, :] * x_ref[i0 + 1, rows, :][None]
                   + w_ref[i0 + 2][:, None, :] * x_ref[i0 + 2, rows, :][None]
                   + w_ref[i0 + 3][:, None, :] * x_ref[i0 + 3, rows, :][None])
            m_ref[...] = m_ref[...] + upd
            return 0

        jax.lax.fori_loop(0, Cin // 4, per_in, 0)
        lane = jax.lax.broadcasted_iota(jnp.int32, (TS, Wl), 1)
        half = ((lane // dout) % 2) == 0

        for co in range(Co):
            band0 = jnp.where(half, m_ref[4 * co + 0], m_ref[4 * co + 1])
            band1 = jnp.where(half, m_ref[4 * co + 2], m_ref[4 * co + 3])
            full = jnp.stack([band0, band1], axis=1).reshape(2 * TS, Wl)
            o_ref[co, pl.ds(oy + 2 * s * TS, 2 * TS), pl.ds(ox, Wl)] = full
        return 0

    jax.lax.fori_loop(0, H // TS, strip, 0)


def _upsample(x, w, b, TS, din, pad_out=False):
    Cmid, Cin = w.shape[0], w.shape[1]
    C, H, Wl = x.shape
    Co = Cmid // 4
    dout = din // 2
    w3 = jnp.broadcast_to(
        w.reshape(Cmid, Cin).transpose(1, 0)[:, :, None], (Cin, Cmid, Wl))
    b2 = jnp.broadcast_to(b[:, None], (Cmid, Wl))
    oshape = ((Co, 2 * _PT + 2 * H, Wl + 2 * dout) if pad_out
              else (Co, 2 * H, Wl))
    fn = functools.partial(_upsample_body, Cin, Cmid, H, Wl, TS, din, pad_out)
    return pl.pallas_call(
        fn,
        out_shape=jax.ShapeDtypeStruct(oshape, _F32),
        scratch_shapes=[pltpu.VMEM((Cmid, TS, Wl), _F32)],
    )(x, w3, b2)


def kernel(x, con11_w, con11_b, con1_w, con1_b, con3_w, con3_b, con5_w,
           con5_b, dm1_gamma, dm1_beta, dm1_lin, dm2_gamma, dm2_beta,
           dm2_lin, up2_w, up2_b, up1_w, up1_b):
    xp = jnp.pad(x[0], ((0, 0), (_PT, _PT), (1, 1)))

    y1 = _conv3x3(xp, con11_w, con11_b, TH=48, mask_inf=True)
    d1 = _pmerge(y1, dm1_gamma, dm1_beta, dm1_lin, TS=24, din=1)
    d2 = _pmerge(d1, dm2_gamma, dm2_beta, dm2_lin, TS=16, din=2)
    u2 = _upsample(d2, up2_w, up2_b, TS=16, din=4, pad_out=True)
    c1 = _conv3x3(u2, con1_w, con1_b, TH=24, din=2)
    u1 = _upsample(c1, up1_w, up1_b, TS=24, din=2, pad_out=True)
    c3 = _conv3x3(u1, con3_w, con3_b, TH=48, pad_out=True)
    c4 = _conv3x3(c3, con3_w, con3_b, TH=48, pad_out=True)
    c5 = _conv3x3(c4, con5_w, con5_b, TH=48)
    return c5.reshape(1, 1, 384, 384)


# speedup vs baseline: 1.1435x; 1.0153x over previous
import functools

import jax
import jax.numpy as jnp
from jax.experimental import pallas as pl
from jax.experimental.pallas import tpu as pltpu

_F32 = jnp.float32
_PT = 8


def _zero_border(o_ref, C, H, Wl, d):
    o_ref[:, 0:_PT, :] = jnp.zeros((C, _PT, Wl + 2 * d), _F32)
    o_ref[:, _PT + H:_PT + H + _PT, :] = jnp.zeros((C, _PT, Wl + 2 * d), _F32)
    o_ref[:, :, 0:d] = jnp.zeros((C, 2 * _PT + H, d), _F32)
    o_ref[:, :, Wl + d:Wl + 2 * d] = jnp.zeros((C, 2 * _PT + H, d), _F32)


def _shl(v, d):
    return jnp.concatenate([v[..., d:], v[..., :d]], axis=-1)


def _shr(v, d):
    return jnp.concatenate([v[..., -d:], v[..., :-d]], axis=-1)


def _conv3x3_body(Cin, Cout, H, Wl, TH, din, pad_out, mask_inf,
                  x_ref, w_ref, b_ref, o_ref, a_ref):
    oy, ox = (_PT, din) if pad_out else (0, 0)
    if pad_out:
        _zero_border(o_ref, Cout, H, Wl, din)

    def strip(s, _):
        r0 = s * TH
        a_ref[...] = jnp.broadcast_to(b_ref[...][:, None, :], (Cout, TH, Wl))

        def per_in(ib, _):
            acc = jnp.zeros((Cout, TH, Wl), _F32)
            for k in range(2):
                i = ib * 2 + k
                slab = x_ref[i, pl.ds(r0, TH + 9), :]
                for dy in range(3):
                    for dx in range(3):
                        wrow = w_ref[i * 9 + dy * 3 + dx]
                        xs = slab[7 + dy:7 + dy + TH,
                                  dx * din:dx * din + Wl]
                        acc = acc + wrow[:, None, :] * xs[None, :, :]
            a_ref[...] = a_ref[...] + acc
            return 0

        jax.lax.fori_loop(0, Cin // 2, per_in, 0)
        v = a_ref[...]
        if mask_inf:
            v = jnp.where(jnp.isinf(v), 0.0, v)
        o_ref[:, pl.ds(oy + r0, TH), pl.ds(ox, Wl)] = v
        return 0

    jax.lax.fori_loop(0, H // TH, strip, 0)


def _conv3x3(xp, w, b, TH, din=1, pad_out=False, mask_inf=False):
    Cout, Cin = w.shape[0], w.shape[1]
    H, Wl = xp.shape[1] - 2 * _PT, xp.shape[2] - 2 * din
    w3 = jnp.broadcast_to(
        w.transpose(1, 2, 3, 0).reshape(Cin * 9, Cout)[:, :, None],
        (Cin * 9, Cout, Wl))
    b2 = jnp.broadcast_to(b[:, None], (Cout, Wl))
    oshape = ((Cout, 2 * _PT + H, Wl + 2 * din) if pad_out
              else (Cout, H, Wl))
    fn = functools.partial(_conv3x3_body, Cin, Cout, H, Wl, TH, din, pad_out,
                           mask_inf)
    return pl.pallas_call(
        fn,
        out_shape=jax.ShapeDtypeStruct(oshape, _F32),
        scratch_shapes=[pltpu.VMEM((Cout, TH, Wl), _F32)],
    )(xp, w3, b2)


def _pmerge_body(C, D, H, Wl, TS, din, x_ref, g_ref, bt_ref, l_ref,
                 o_ref, n_ref):
    inv = 1.0 / (4 * C)

    def strip(s, _):
        xr = x_ref[:, pl.ds(2 * s * TS, 2 * TS), :].reshape(C, TS, 2, Wl)
        A = jnp.concatenate([xr[:, :, 0, :], xr[:, :, 1, :]], axis=0)
        lane = jax.lax.broadcasted_iota(jnp.int32, (TS, Wl), 1)
        m = ((lane // din) % 2) == 0

        def pair(v):
            return v + jnp.where(m, _shl(v, din), _shr(v, din))

        s1 = jnp.sum(A, axis=0)
        q1 = jnp.sum(A * A, axis=0)
        mu = pair(s1) * inv
        var = pair(q1) * inv - mu * mu
        rstd = jax.lax.rsqrt(var + 1e-5)
        n_ref[...] = ((A - mu[None]) * rstd[None] * g_ref[...][:, None, :]
                      + bt_ref[...][:, None, :])
        rows = pl.ds(s * TS, TS)
        o_ref[:, rows, :] = jnp.zeros((D, TS, Wl), _F32)

        def per_in(gb, _):
            g0 = gb * 4
            upd = (l_ref[g0][:, None, :] * n_ref[g0][None]
                   + l_ref[g0 + 1][:, None, :] * n_ref[g0 + 1][None]
                   + l_ref[g0 + 2][:, None, :] * n_ref[g0 + 2][None]
                   + l_ref[g0 + 3][:, None, :] * n_ref[g0 + 3][None])
            o_ref[:, rows, :] = o_ref[:, rows, :] + upd
            return 0

        jax.lax.fori_loop(0, (2 * C) // 4, per_in, 0)
        t = o_ref[:, rows, :]
        o_ref[:, rows, :] = t + jnp.where(m[None], _shl(t, din), _shr(t, din))
        return 0

    jax.lax.fori_loop(0, (H // 2) // TS, strip, 0)


def _pmerge(x, gamma, beta, lin, TS, din):
    C, H, Wl = x.shape
    D = lin.shape[1]
    pat = ((jnp.arange(Wl) // din) % 2) == 0
    g4 = gamma.reshape(2, 2, C)
    gfull = jnp.where(pat[None, :], g4[0].reshape(2 * C)[:, None],
                      g4[1].reshape(2 * C)[:, None])
    b4 = beta.reshape(2, 2, C)
    btfull = jnp.where(pat[None, :], b4[0].reshape(2 * C)[:, None],
                       b4[1].reshape(2 * C)[:, None])
    l4 = lin.reshape(2, 2, C, D)
    lfull = jnp.where(pat[None, None, :],
                      l4[0].reshape(2 * C, D)[:, :, None],
                      l4[1].reshape(2 * C, D)[:, :, None])
    fn = functools.partial(_pmerge_body, C, D, H, Wl, TS, din)
    return pl.pallas_call(
        fn,
        out_shape=jax.ShapeDtypeStruct((D, H // 2, Wl), _F32),
        scratch_shapes=[pltpu.VMEM((2 * C, TS, Wl), _F32)],
    )(x, gfull, btfull, lfull)


def _upsample_body(Cin, Cmid, H, Wl, TS, din, pad_out,
                   x_ref, w_ref, b_ref, o_ref, m_ref):
    Co = Cmid // 4
    dout = din // 2
    oy, ox = (_PT, dout) if pad_out else (0, 0)
    if pad_out:
        _zero_border(o_ref, Co, 2 * H, Wl, dout)

    def strip(s, _):
        rows = pl.ds(s * TS, TS)
        m_ref[...] = jnp.broadcast_to(b_ref[...][:, None, :], (Cmid, TS, Wl))

        def per_in(ib, _):
            i0 = ib * 4
            upd = (w_ref[i0][:, None, :] * x_ref[i0, rows, :][None]
                   + w_ref[i0 + 1][:, None, :] * x_ref[i0 + 1, rows, :][None]
                   + w_ref[i0 + 2][:, None, :] * x_ref[i0 + 2, rows, :][None]
                   + w_ref[i0 + 3][:, None, :] * x_ref[i0 + 3, rows, :][None])
            m_ref[...] = m_ref[...] + upd
            return 0

        jax.lax.fori_loop(0, Cin // 4, per_in, 0)
        lane = jax.lax.broadcasted_iota(jnp.int32, (TS, Wl), 1)
        half = ((lane // dout) % 2) == 0

        for co in range(Co):
            band0 = jnp.where(half, m_ref[4 * co + 0], m_ref[4 * co + 1])
            band1 = jnp.where(half, m_ref[4 * co + 2], m_ref[4 * co + 3])
            full = jnp.stack([band0, band1], axis=1).reshape(2 * TS, Wl)
            o_ref[co, pl.ds(oy + 2 * s * TS, 2 * TS), pl.ds(ox, Wl)] = full
        return 0

    jax.lax.fori_loop(0, H // TS, strip, 0)


def _upsample(x, w, b, TS, din, pad_out=False):
    Cmid, Cin = w.shape[0], w.shape[1]
    C, H, Wl = x.shape
    Co = Cmid // 4
    dout = din // 2
    w3 = jnp.broadcast_to(
        w.reshape(Cmid, Cin).transpose(1, 0)[:, :, None], (Cin, Cmid, Wl))
    b2 = jnp.broadcast_to(b[:, None], (Cmid, Wl))
    oshape = ((Co, 2 * _PT + 2 * H, Wl + 2 * dout) if pad_out
              else (Co, 2 * H, Wl))
    fn = functools.partial(_upsample_body, Cin, Cmid, H, Wl, TS, din, pad_out)
    return pl.pallas_call(
        fn,
        out_shape=jax.ShapeDtypeStruct(oshape, _F32),
        scratch_shapes=[pltpu.VMEM((Cmid, TS, Wl), _F32)],
    )(x, w3, b2)


def kernel(x, con11_w, con11_b, con1_w, con1_b, con3_w, con3_b, con5_w,
           con5_b, dm1_gamma, dm1_beta, dm1_lin, dm2_gamma, dm2_beta,
           dm2_lin, up2_w, up2_b, up1_w, up1_b):
    xp = jnp.pad(x[0], ((0, 0), (_PT, _PT), (1, 1)))

    y1 = _conv3x3(xp, con11_w, con11_b, TH=48, mask_inf=True)
    d1 = _pmerge(y1, dm1_gamma, dm1_beta, dm1_lin, TS=24, din=1)
    d2 = _pmerge(d1, dm2_gamma, dm2_beta, dm2_lin, TS=16, din=2)
    u2 = _upsample(d2, up2_w, up2_b, TS=16, din=4, pad_out=True)
    c1 = _conv3x3(u2, con1_w, con1_b, TH=24, din=2)
    u1 = _upsample(c1, up1_w, up1_b, TS=24, din=2, pad_out=True)
    c3 = _conv3x3(u1, con3_w, con3_b, TH=48, pad_out=True)
    c4 = _conv3x3(c3, con3_w, con3_b, TH=48, pad_out=True)
    c5 = _conv3x3(c4, con5_w, con5_b, TH=48)
    return c5.reshape(1, 1, 384, 384)


# speedup vs baseline: 1.1527x; 1.0081x over previous
import functools

import jax
import jax.numpy as jnp
from jax.experimental import pallas as pl
from jax.experimental.pallas import tpu as pltpu

_F32 = jnp.float32
_PT = 8


def _zero_border(o_ref, C, H, Wl, d):
    o_ref[:, 0:_PT, :] = jnp.zeros((C, _PT, Wl + 2 * d), _F32)
    o_ref[:, _PT + H:_PT + H + _PT, :] = jnp.zeros((C, _PT, Wl + 2 * d), _F32)
    o_ref[:, :, 0:d] = jnp.zeros((C, 2 * _PT + H, d), _F32)
    o_ref[:, :, Wl + d:Wl + 2 * d] = jnp.zeros((C, 2 * _PT + H, d), _F32)


def _shl(v, d):
    return jnp.concatenate([v[..., d:], v[..., :d]], axis=-1)


def _shr(v, d):
    return jnp.concatenate([v[..., -d:], v[..., :-d]], axis=-1)


def _conv3x3_body(Cin, Cout, H, Wl, TH, din, pad_out, mask_inf,
                  x_ref, w_ref, b_ref, o_ref, a_ref):
    oy, ox = (_PT, din) if pad_out else (0, 0)
    if pad_out:
        _zero_border(o_ref, Cout, H, Wl, din)

    def strip(s, _):
        r0 = s * TH
        a_ref[...] = jnp.broadcast_to(b_ref[...][:, None, :], (Cout, TH, Wl))

        def per_in(ib, _):
            acc = jnp.zeros((Cout, TH, Wl), _F32)
            for k in range(4):
                i = ib * 4 + k
                slab = x_ref[i, pl.ds(r0, TH + 9), :]
                for dy in range(3):
                    for dx in range(3):
                        wrow = w_ref[i * 9 + dy * 3 + dx]
                        xs = slab[7 + dy:7 + dy + TH,
                                  dx * din:dx * din + Wl]
                        acc = acc + wrow[:, None, :] * xs[None, :, :]
            a_ref[...] = a_ref[...] + acc
            return 0

        jax.lax.fori_loop(0, Cin // 4, per_in, 0)
        v = a_ref[...]
        if mask_inf:
            v = jnp.where(jnp.isinf(v), 0.0, v)
        o_ref[:, pl.ds(oy + r0, TH), pl.ds(ox, Wl)] = v
        return 0

    jax.lax.fori_loop(0, H // TH, strip, 0)


def _conv3x3(xp, w, b, TH, din=1, pad_out=False, mask_inf=False):
    Cout, Cin = w.shape[0], w.shape[1]
    H, Wl = xp.shape[1] - 2 * _PT, xp.shape[2] - 2 * din
    w3 = jnp.broadcast_to(
        w.transpose(1, 2, 3, 0).reshape(Cin * 9, Cout)[:, :, None],
        (Cin * 9, Cout, Wl))
    b2 = jnp.broadcast_to(b[:, None], (Cout, Wl))
    oshape = ((Cout, 2 * _PT + H, Wl + 2 * din) if pad_out
              else (Cout, H, Wl))
    fn = functools.partial(_conv3x3_body, Cin, Cout, H, Wl, TH, din, pad_out,
                           mask_inf)
    return pl.pallas_call(
        fn,
        out_shape=jax.ShapeDtypeStruct(oshape, _F32),
        scratch_shapes=[pltpu.VMEM((Cout, TH, Wl), _F32)],
    )(xp, w3, b2)


def _pmerge_body(C, D, H, Wl, TS, din, x_ref, g_ref, bt_ref, l_ref,
                 o_ref, n_ref):
    inv = 1.0 / (4 * C)

    def strip(s, _):
        xr = x_ref[:, pl.ds(2 * s * TS, 2 * TS), :].reshape(C, TS, 2, Wl)
        A = jnp.concatenate([xr[:, :, 0, :], xr[:, :, 1, :]], axis=0)
        lane = jax.lax.broadcasted_iota(jnp.int32, (TS, Wl), 1)
        m = ((lane // din) % 2) == 0

        def pair(v):
            return v + jnp.where(m, _shl(v, din), _shr(v, din))

        s1 = jnp.sum(A, axis=0)
        q1 = jnp.sum(A * A, axis=0)
        mu = pair(s1) * inv
        var = pair(q1) * inv - mu * mu
        rstd = jax.lax.rsqrt(var + 1e-5)
        n_ref[...] = ((A - mu[None]) * rstd[None] * g_ref[...][:, None, :]
                      + bt_ref[...][:, None, :])
        rows = pl.ds(s * TS, TS)
        o_ref[:, rows, :] = jnp.zeros((D, TS, Wl), _F32)

        def per_in(gb, _):
            g0 = gb * 4
            upd = (l_ref[g0][:, None, :] * n_ref[g0][None]
                   + l_ref[g0 + 1][:, None, :] * n_ref[g0 + 1][None]
                   + l_ref[g0 + 2][:, None, :] * n_ref[g0 + 2][None]
                   + l_ref[g0 + 3][:, None, :] * n_ref[g0 + 3][None])
            o_ref[:, rows, :] = o_ref[:, rows, :] + upd
            return 0

        jax.lax.fori_loop(0, (2 * C) // 4, per_in, 0)
        t = o_ref[:, rows, :]
        o_ref[:, rows, :] = t + jnp.where(m[None], _shl(t, din), _shr(t, din))
        return 0

    jax.lax.fori_loop(0, (H // 2) // TS, strip, 0)


def _pmerge(x, gamma, beta, lin, TS, din):
    C, H, Wl = x.shape
    D = lin.shape[1]
    pat = ((jnp.arange(Wl) // din) % 2) == 0
    g4 = gamma.reshape(2, 2, C)
    gfull = jnp.where(pat[None, :], g4[0].reshape(2 * C)[:, None],
                      g4[1].reshape(2 * C)[:, None])
    b4 = beta.reshape(2, 2, C)
    btfull = jnp.where(pat[None, :], b4[0].reshape(2 * C)[:, None],
                       b4[1].reshape(2 * C)[:, None])
    l4 = lin.reshape(2, 2, C, D)
    lfull = jnp.where(pat[None, None, :],
                      l4[0].reshape(2 * C, D)[:, :, None],
                      l4[1].reshape(2 * C, D)[:, :, None])
    fn = functools.partial(_pmerge_body, C, D, H, Wl, TS, din)
    return pl.pallas_call(
        fn,
        out_shape=jax.ShapeDtypeStruct((D, H // 2, Wl), _F32),
        scratch_shapes=[pltpu.VMEM((2 * C, TS, Wl), _F32)],
    )(x, gfull, btfull, lfull)


def _upsample_body(Cin, Cmid, H, Wl, TS, din, pad_out,
                   x_ref, w_ref, b_ref, o_ref, m_ref):
    Co = Cmid // 4
    dout = din // 2
    oy, ox = (_PT, dout) if pad_out else (0, 0)
    if pad_out:
        _zero_border(o_ref, Co, 2 * H, Wl, dout)

    def strip(s, _):
        rows = pl.ds(s * TS, TS)
        m_ref[...] = jnp.broadcast_to(b_ref[...][:, None, :], (Cmid, TS, Wl))

        def per_in(ib, _):
            i0 = ib * 4
            upd = (w_ref[i0][:, None, :] * x_ref[i0, rows, :][None]
                   + w_ref[i0 + 1][:, None, :] * x_ref[i0 + 1, rows, :][None]
                   + w_ref[i0 + 2][:, None, :] * x_ref[i0 + 2, rows, :][None]
                   + w_ref[i0 + 3][:, None, :] * x_ref[i0 + 3, rows, :][None])
            m_ref[...] = m_ref[...] + upd
            return 0

        jax.lax.fori_loop(0, Cin // 4, per_in, 0)
        lane = jax.lax.broadcasted_iota(jnp.int32, (TS, Wl), 1)
        half = ((lane // dout) % 2) == 0

        for co in range(Co):
            band0 = jnp.where(half, m_ref[4 * co + 0], m_ref[4 * co + 1])
            band1 = jnp.where(half, m_ref[4 * co + 2], m_ref[4 * co + 3])
            full = jnp.stack([band0, band1], axis=1).reshape(2 * TS, Wl)
            o_ref[co, pl.ds(oy + 2 * s * TS, 2 * TS), pl.ds(ox, Wl)] = full
        return 0

    jax.lax.fori_loop(0, H // TS, strip, 0)


def _upsample(x, w, b, TS, din, pad_out=False):
    Cmid, Cin = w.shape[0], w.shape[1]
    C, H, Wl = x.shape
    Co = Cmid // 4
    dout = din // 2
    w3 = jnp.broadcast_to(
        w.reshape(Cmid, Cin).transpose(1, 0)[:, :, None], (Cin, Cmid, Wl))
    b2 = jnp.broadcast_to(b[:, None], (Cmid, Wl))
    oshape = ((Co, 2 * _PT + 2 * H, Wl + 2 * dout) if pad_out
              else (Co, 2 * H, Wl))
    fn = functools.partial(_upsample_body, Cin, Cmid, H, Wl, TS, din, pad_out)
    return pl.pallas_call(
        fn,
        out_shape=jax.ShapeDtypeStruct(oshape, _F32),
        scratch_shapes=[pltpu.VMEM((Cmid, TS, Wl), _F32)],
    )(x, w3, b2)


def kernel(x, con11_w, con11_b, con1_w, con1_b, con3_w, con3_b, con5_w,
           con5_b, dm1_gamma, dm1_beta, dm1_lin, dm2_gamma, dm2_beta,
           dm2_lin, up2_w, up2_b, up1_w, up1_b):
    xp = jnp.pad(x[0], ((0, 0), (_PT, _PT), (1, 1)))

    y1 = _conv3x3(xp, con11_w, con11_b, TH=48, mask_inf=True)
    d1 = _pmerge(y1, dm1_gamma, dm1_beta, dm1_lin, TS=24, din=1)
    d2 = _pmerge(d1, dm2_gamma, dm2_beta, dm2_lin, TS=16, din=2)
    u2 = _upsample(d2, up2_w, up2_b, TS=16, din=4, pad_out=True)
    c1 = _conv3x3(u2, con1_w, con1_b, TH=24, din=2)
    u1 = _upsample(c1, up1_w, up1_b, TS=24, din=2, pad_out=True)
    c3 = _conv3x3(u1, con3_w, con3_b, TH=48, pad_out=True)
    c4 = _conv3x3(c3, con3_w, con3_b, TH=48, pad_out=True)
    c5 = _conv3x3(c4, con5_w, con5_b, TH=48)
    return c5.reshape(1, 1, 384, 384)
